# baseline (speedup 1.0000x reference)
import functools

import jax
import jax.numpy as jnp
from jax.experimental import pallas as pl
from jax.experimental.pallas import tpu as pltpu

BEAMS = 128
HID = 1024
VOCAB = 100000
K = 8
BV = 2000
NBLK = VOCAB // BV
LANES = 128
NSLOT = BV // 8
NGRP = 16

NEG = -1e30
BIGI = 2**30


def _bitonic_merge_desc(xs):
    n = len(xs)
    if n == 1:
        return xs
    half = n // 2
    hi = [jnp.maximum(xs[i], xs[i + half]) for i in range(half)]
    lo = [jnp.minimum(xs[i], xs[i + half]) for i in range(half)]
    return _bitonic_merge_desc(hi) + _bitonic_merge_desc(lo)


def _merge_desc(a, b):
    return _bitonic_merge_desc(a + b[::-1])


def _merge_top8(a, b):
    m = [jnp.maximum(a[i], b[7 - i]) for i in range(8)]
    return _bitonic_merge_desc(m)


def _block_top8(tiles):
    lists = [[t] for t in tiles]
    while len(lists) > 2:
        lists = [_merge_desc(lists[t], lists[t + 1])
                 for t in range(0, len(lists), 2)]
    return _merge_top8(lists[0], lists[1])


def _bitonic_merge_desc_kv(vs, ids):
    n = len(vs)
    if n == 1:
        return vs, ids
    half = n // 2
    hv, hi, lv, li = [], [], [], []
    for i in range(half):
        av, ai, bv, bi = vs[i], ids[i], vs[i + half], ids[i + half]
        c = bv > av
        hv.append(jnp.maximum(av, bv))
        hi.append(jnp.where(c, bi, ai))
        lv.append(jnp.minimum(av, bv))
        li.append(jnp.where(c, ai, bi))
    rhv, rhi = _bitonic_merge_desc_kv(hv, hi)
    rlv, rli = _bitonic_merge_desc_kv(lv, li)
    return rhv + rlv, rhi + rli


def _merge_top8_kv(av, ai, bv, bi):
    mv, mi = [], []
    for i in range(8):
        x, xi_, y, yi = av[i], ai[i], bv[7 - i], bi[7 - i]
        c = y > x
        mv.append(jnp.maximum(x, y))
        mi.append(jnp.where(c, yi, xi_))
    return _bitonic_merge_desc_kv(mv, mi)


def _step(hidt_ref, wt_ref, b_ref, prev_ref,
          vals_out, ids_out,
          m_scr, s_scr, pv_scr, pi_scr):
    j = pl.program_id(0)

    @pl.when(j == 0)
    def _init():
        m_scr[...] = jnp.full((1, LANES), NEG, jnp.float32)
        s_scr[...] = jnp.zeros((1, LANES), jnp.float32)
        pv_scr[...] = jnp.full((8 * K, LANES), NEG, jnp.float32)
        pi_scr[...] = jnp.full((8 * K, LANES), BIGI, jnp.int32)

    x_row = jax.lax.dot_general(
        hidt_ref[...], wt_ref[...], (((0,), (1,)), ((), ())),
        preferred_element_type=jnp.float32,
        precision=jax.lax.Precision.HIGHEST,
    ) + b_ref[...].reshape(1, BV)
    xt = x_row.T

    m_old = m_scr[...]
    bm = jnp.max(xt, axis=0, keepdims=True)
    m_new = jnp.maximum(m_old, bm)
    s_scr[...] = (s_scr[...] * jnp.exp(m_old - m_new)
                  + jnp.sum(jnp.exp(xt - m_new), axis=0, keepdims=True))
    m_scr[...] = m_new

    subl = jax.lax.broadcasted_iota(jnp.int32, (8, LANES), 0)
    negslot = jnp.full((8, LANES), NEG, jnp.float32)
    gv, gi = [], []
    for g in range(NGRP):
        tiles = []
        for t in range(16):
            s = g * 16 + t
            if s < NSLOT:
                xi = jax.lax.bitcast_convert_type(
                    xt[s * 8:(s + 1) * 8, :], jnp.int32)
                tiles.append(jax.lax.bitcast_convert_type(
                    (xi & -16) | t, jnp.float32))
            else:
                tiles.append(negslot)
        blk = _block_top8(tiles)
        bv_, bi_ = [], []
        for r in range(K):
            y = jax.lax.bitcast_convert_type(blk[r], jnp.int32)
            slot = (y & 15) + g * 16
            bi_.append(slot * 8 + subl + j * BV)
            bv_.append(jax.lax.bitcast_convert_type(y & -16, jnp.float32))
        gv.append(bv_)
        gi.append(bi_)

    while len(gv) > 1:
        nv, ni = [], []
        for t in range(0, len(gv), 2):
            mv, mi = _merge_top8_kv(gv[t], gi[t], gv[t + 1], gi[t + 1])
            nv.append(mv)
            ni.append(mi)
        gv, gi = nv, ni

    pv = [pv_scr[p * 8:(p + 1) * 8, :] for p in range(K)]
    pi = [pi_scr[p * 8:(p + 1) * 8, :] for p in range(K)]
    nv, ni = _merge_top8_kv(pv, pi, gv[0], gi[0])
    for p in range(K):
        pv_scr[p * 8:(p + 1) * 8, :] = nv[p]
        pi_scr[p * 8:(p + 1) * 8, :] = ni[p]

    @pl.when(j == NBLK - 1)
    def _finalize():
        x = pv_scr[...]
        ids = pi_scr[...]
        tvs, tis = [], []
        for _ in range(K):
            m = jnp.max(x, axis=0, keepdims=True)
            sel = jnp.min(jnp.where(x == m, ids, BIGI), axis=0,
                          keepdims=True)
            tvs.append(m)
            tis.append(sel)
            x = jnp.where(ids == sel, NEG, x)
        tv = jnp.concatenate(tvs, axis=0)
        ti = jnp.concatenate(tis, axis=0)
        lse = m_scr[...] + jnp.log(s_scr[...])
        sc = prev_ref[...] + tv - lse
        beam = jax.lax.broadcasted_iota(jnp.int32, (K, LANES), 1)
        flat = beam * VOCAB + ti
        ocol = jax.lax.broadcasted_iota(jnp.int32, (1, K), 1)
        ov = jnp.zeros((1, K), jnp.float32)
        oi = jnp.zeros((1, K), jnp.int32)
        for r in range(K):
            m = jnp.max(sc, axis=(0, 1), keepdims=True)
            chosen = jnp.min(jnp.where(sc == m, flat, BIGI),
                             axis=(0, 1), keepdims=True)
            ov = jnp.where(ocol == r, m, ov)
            oi = jnp.where(ocol == r, chosen, oi)
            sc = jnp.where(flat == chosen, NEG, sc)
        vals_out[...] = ov
        ids_out[...] = oi


@functools.partial(jax.jit, static_argnames=())
def kernel(hidden, W, b, prev_log_probs):
    wt = W.T
    hidt = hidden.T
    b2 = b.reshape(NBLK, 1, BV)
    prev2 = prev_log_probs.reshape(1, BEAMS)
    vals, flat = pl.pallas_call(
        _step,
        grid=(NBLK,),
        in_specs=[
            pl.BlockSpec((HID, BEAMS), lambda j: (0, 0)),
            pl.BlockSpec((BV, HID), lambda j: (j, 0)),
            pl.BlockSpec((1, 1, BV), lambda j: (j, 0, 0)),
            pl.BlockSpec((1, BEAMS), lambda j: (0, 0)),
        ],
        out_specs=[
            pl.BlockSpec((1, K), lambda j: (0, 0)),
            pl.BlockSpec((1, K), lambda j: (0, 0)),
        ],
        out_shape=[
            jax.ShapeDtypeStruct((1, K), jnp.float32),
            jax.ShapeDtypeStruct((1, K), jnp.int32),
        ],
        scratch_shapes=[
            pltpu.VMEM((1, LANES), jnp.float32),
            pltpu.VMEM((1, LANES), jnp.float32),
            pltpu.VMEM((8 * K, LANES), jnp.float32),
            pltpu.VMEM((8 * K, LANES), jnp.int32),
        ],
        compiler_params=pltpu.CompilerParams(
            dimension_semantics=("arbitrary",),
        ),
    )(hidt, wt, b2, prev2)
    vals = vals.reshape(K)
    flat = flat.reshape(K)
    beam_ids = flat // VOCAB
    token_ids = flat % VOCAB
    return vals, beam_ids, token_ids


# speedup vs baseline: 1.1237x; 1.1237x over previous
import functools

import jax
import jax.numpy as jnp
from jax.experimental import pallas as pl
from jax.experimental.pallas import tpu as pltpu

BEAMS = 128
HID = 1024
VOCAB = 100000
K = 8
BV = 2000
NBLK = VOCAB // BV
LANES = 128
NSLOT = BV // 8
NGRP = 16

NEG = -1e30
BIGI = 2**30


def _bitonic_merge_desc(xs):
    n = len(xs)
    if n == 1:
        return xs
    half = n // 2
    hi = [jnp.maximum(xs[i], xs[i + half]) for i in range(half)]
    lo = [jnp.minimum(xs[i], xs[i + half]) for i in range(half)]
    return _bitonic_merge_desc(hi) + _bitonic_merge_desc(lo)


def _merge_desc(a, b):
    return _bitonic_merge_desc(a + b[::-1])


def _merge_top8(a, b):
    m = [jnp.maximum(a[i], b[7 - i]) for i in range(8)]
    return _bitonic_merge_desc(m)


def _block_top8(tiles):
    lists = [[t] for t in tiles]
    while len(lists) > 2:
        lists = [_merge_desc(lists[t], lists[t + 1])
                 for t in range(0, len(lists), 2)]
    return _merge_top8(lists[0], lists[1])


def _bitonic_merge_desc_kv(vs, ids):
    n = len(vs)
    if n == 1:
        return vs, ids
    half = n // 2
    hv, hi, lv, li = [], [], [], []
    for i in range(half):
        av, ai, bv, bi = vs[i], ids[i], vs[i + half], ids[i + half]
        c = bv > av
        hv.append(jnp.maximum(av, bv))
        hi.append(jnp.where(c, bi, ai))
        lv.append(jnp.minimum(av, bv))
        li.append(jnp.where(c, ai, bi))
    rhv, rhi = _bitonic_merge_desc_kv(hv, hi)
    rlv, rli = _bitonic_merge_desc_kv(lv, li)
    return rhv + rlv, rhi + rli


def _merge_top8_kv(av, ai, bv, bi):
    mv, mi = [], []
    for i in range(8):
        x, xi_, y, yi = av[i], ai[i], bv[7 - i], bi[7 - i]
        c = y > x
        mv.append(jnp.maximum(x, y))
        mi.append(jnp.where(c, yi, xi_))
    return _bitonic_merge_desc_kv(mv, mi)


def _step(hidt_ref, wt_ref, b_ref, prev_ref,
          vals_out, ids_out,
          m_scr, s_scr, pv_scr, pi_scr):
    j = pl.program_id(0)

    @pl.when(j == 0)
    def _init():
        m_scr[...] = jnp.full((1, LANES), NEG, jnp.float32)
        s_scr[...] = jnp.zeros((1, LANES), jnp.float32)
        pv_scr[...] = jnp.full((8 * K, LANES), NEG, jnp.float32)
        pi_scr[...] = jnp.full((8 * K, LANES), BIGI, jnp.int32)

    ones = jnp.full((1, LANES), 1.0, jnp.float32)
    bias = jax.lax.dot_general(
        b_ref[...].reshape(1, BV), ones, (((0,), (0,)), ((), ())),
        preferred_element_type=jnp.float32,
        precision=jax.lax.Precision.HIGHEST,
    )
    xt = jax.lax.dot_general(
        wt_ref[...], hidt_ref[...], (((1,), (0,)), ((), ())),
        preferred_element_type=jnp.float32,
        precision=jax.lax.Precision.HIGHEST,
    ) + bias

    m_old = m_scr[...]
    bm = jnp.max(xt, axis=0, keepdims=True)
    m_new = jnp.maximum(m_old, bm)
    s_scr[...] = (s_scr[...] * jnp.exp(m_old - m_new)
                  + jnp.sum(jnp.exp(xt - m_new), axis=0, keepdims=True))
    m_scr[...] = m_new

    subl = jax.lax.broadcasted_iota(jnp.int32, (8, LANES), 0)
    negslot = jnp.full((8, LANES), NEG, jnp.float32)
    gv, gi = [], []
    for g in range(NGRP):
        tiles = []
        for t in range(16):
            s = g * 16 + t
            if s < NSLOT:
                xi = jax.lax.bitcast_convert_type(
                    xt[s * 8:(s + 1) * 8, :], jnp.int32)
                tiles.append(jax.lax.bitcast_convert_type(
                    (xi & -16) | t, jnp.float32))
            else:
                tiles.append(negslot)
        blk = _block_top8(tiles)
        bv_, bi_ = [], []
        for r in range(K):
            y = jax.lax.bitcast_convert_type(blk[r], jnp.int32)
            slot = (y & 15) + g * 16
            bi_.append(slot * 8 + subl + j * BV)
            bv_.append(jax.lax.bitcast_convert_type(y & -16, jnp.float32))
        gv.append(bv_)
        gi.append(bi_)

    while len(gv) > 1:
        nv, ni = [], []
        for t in range(0, len(gv), 2):
            mv, mi = _merge_top8_kv(gv[t], gi[t], gv[t + 1], gi[t + 1])
            nv.append(mv)
            ni.append(mi)
        gv, gi = nv, ni

    pv = [pv_scr[p * 8:(p + 1) * 8, :] for p in range(K)]
    pi = [pi_scr[p * 8:(p + 1) * 8, :] for p in range(K)]
    nv, ni = _merge_top8_kv(pv, pi, gv[0], gi[0])
    for p in range(K):
        pv_scr[p * 8:(p + 1) * 8, :] = nv[p]
        pi_scr[p * 8:(p + 1) * 8, :] = ni[p]

    @pl.when(j == NBLK - 1)
    def _finalize():
        x = pv_scr[...]
        ids = pi_scr[...]
        tvs, tis = [], []
        for _ in range(K):
            m = jnp.max(x, axis=0, keepdims=True)
            sel = jnp.min(jnp.where(x == m, ids, BIGI), axis=0,
                          keepdims=True)
            tvs.append(m)
            tis.append(sel)
            x = jnp.where(ids == sel, NEG, x)
        tv = jnp.concatenate(tvs, axis=0)
        ti = jnp.concatenate(tis, axis=0)
        lse = m_scr[...] + jnp.log(s_scr[...])
        sc = prev_ref[...] + tv - lse
        beam = jax.lax.broadcasted_iota(jnp.int32, (K, LANES), 1)
        flat = beam * VOCAB + ti
        ocol = jax.lax.broadcasted_iota(jnp.int32, (1, K), 1)
        ov = jnp.zeros((1, K), jnp.float32)
        oi = jnp.zeros((1, K), jnp.int32)
        for r in range(K):
            m = jnp.max(sc, axis=(0, 1), keepdims=True)
            chosen = jnp.min(jnp.where(sc == m, flat, BIGI),
                             axis=(0, 1), keepdims=True)
            ov = jnp.where(ocol == r, m, ov)
            oi = jnp.where(ocol == r, chosen, oi)
            sc = jnp.where(flat == chosen, NEG, sc)
        vals_out[...] = ov
        ids_out[...] = oi


@functools.partial(jax.jit, static_argnames=())
def kernel(hidden, W, b, prev_log_probs):
    wt = W.T
    hidt = hidden.T
    b2 = b.reshape(NBLK, 1, BV)
    prev2 = prev_log_probs.reshape(1, BEAMS)
    vals, flat = pl.pallas_call(
        _step,
        grid=(NBLK,),
        in_specs=[
            pl.BlockSpec((HID, BEAMS), lambda j: (0, 0)),
            pl.BlockSpec((BV, HID), lambda j: (j, 0)),
            pl.BlockSpec((1, 1, BV), lambda j: (j, 0, 0)),
            pl.BlockSpec((1, BEAMS), lambda j: (0, 0)),
        ],
        out_specs=[
            pl.BlockSpec((1, K), lambda j: (0, 0)),
            pl.BlockSpec((1, K), lambda j: (0, 0)),
        ],
        out_shape=[
            jax.ShapeDtypeStruct((1, K), jnp.float32),
            jax.ShapeDtypeStruct((1, K), jnp.int32),
        ],
        scratch_shapes=[
            pltpu.VMEM((1, LANES), jnp.float32),
            pltpu.VMEM((1, LANES), jnp.float32),
            pltpu.VMEM((8 * K, LANES), jnp.float32),
            pltpu.VMEM((8 * K, LANES), jnp.int32),
        ],
        compiler_params=pltpu.CompilerParams(
            dimension_semantics=("arbitrary",),
        ),
    )(hidt, wt, b2, prev2)
    vals = vals.reshape(K)
    flat = flat.reshape(K)
    beam_ids = flat // VOCAB
    token_ids = flat % VOCAB
    return vals, beam_ids, token_ids


# speedup vs baseline: 1.1877x; 1.0569x over previous
import functools

import jax
import jax.numpy as jnp
from jax.experimental import pallas as pl
from jax.experimental.pallas import tpu as pltpu

BEAMS = 128
HID = 1024
VOCAB = 100000
K = 8
BV = 2000
NBLK = VOCAB // BV
LANES = 128
NSLOT = BV // 8
NGRP = (NSLOT + 15) // 16

NEG = -1e30
BIGI = 2**30


def _bitonic_merge_desc(xs):
    n = len(xs)
    if n == 1:
        return xs
    half = n // 2
    hi = [jnp.maximum(xs[i], xs[i + half]) for i in range(half)]
    lo = [jnp.minimum(xs[i], xs[i + half]) for i in range(half)]
    return _bitonic_merge_desc(hi) + _bitonic_merge_desc(lo)


def _merge_desc(a, b):
    return _bitonic_merge_desc(a + b[::-1])


def _merge_top8(a, b):
    m = [jnp.maximum(a[i], b[7 - i]) for i in range(8)]
    return _bitonic_merge_desc(m)


def _block_top8(tiles):
    lists = [[t] for t in tiles]
    while len(lists) > 2:
        lists = [_merge_desc(lists[t], lists[t + 1])
                 for t in range(0, len(lists), 2)]
    return _merge_top8(lists[0], lists[1])


def _bitonic_merge_desc_kv(vs, ids):
    n = len(vs)
    if n == 1:
        return vs, ids
    half = n // 2
    hv, hi, lv, li = [], [], [], []
    for i in range(half):
        av, ai, bv, bi = vs[i], ids[i], vs[i + half], ids[i + half]
        c = bv > av
        hv.append(jnp.maximum(av, bv))
        hi.append(jnp.where(c, bi, ai))
        lv.append(jnp.minimum(av, bv))
        li.append(jnp.where(c, ai, bi))
    rhv, rhi = _bitonic_merge_desc_kv(hv, hi)
    rlv, rli = _bitonic_merge_desc_kv(lv, li)
    return rhv + rlv, rhi + rli


def _merge_top8_kv(av, ai, bv, bi):
    mv, mi = [], []
    for i in range(8):
        x, xi_, y, yi = av[i], ai[i], bv[7 - i], bi[7 - i]
        c = y > x
        mv.append(jnp.maximum(x, y))
        mi.append(jnp.where(c, yi, xi_))
    return _bitonic_merge_desc_kv(mv, mi)


def _step(hidt_ref, wt_ref, b_ref, prev_ref,
          vals_out, ids_out,
          m_scr, s_scr, pv_scr, pi_scr):
    j = pl.program_id(0)

    @pl.when(j == 0)
    def _init():
        m_scr[...] = jnp.full((1, LANES), NEG, jnp.float32)
        s_scr[...] = jnp.zeros((1, LANES), jnp.float32)
        pv_scr[...] = jnp.full((8 * K, LANES), NEG, jnp.float32)
        pi_scr[...] = jnp.full((8 * K, LANES), BIGI, jnp.int32)

    bias = b_ref[...].reshape(1, BV).T
    xt = jax.lax.dot_general(
        wt_ref[...], hidt_ref[...], (((1,), (0,)), ((), ())),
        preferred_element_type=jnp.float32,
        precision=jax.lax.Precision.HIGHEST,
    ) + bias

    m_old = m_scr[...]
    bm = jnp.max(xt, axis=0, keepdims=True)
    m_new = jnp.maximum(m_old, bm)
    s_scr[...] = (s_scr[...] * jnp.exp(m_old - m_new)
                  + jnp.sum(jnp.exp(xt - m_new), axis=0, keepdims=True))
    m_scr[...] = m_new

    subl = jax.lax.broadcasted_iota(jnp.int32, (8, LANES), 0)
    negslot = jnp.full((8, LANES), NEG, jnp.float32)
    gv, gi = [], []
    for g in range(NGRP):
        tiles = []
        for t in range(16):
            s = g * 16 + t
            if s < NSLOT:
                xi = jax.lax.bitcast_convert_type(
                    xt[s * 8:(s + 1) * 8, :], jnp.int32)
                tiles.append(jax.lax.bitcast_convert_type(
                    (xi & -16) | t, jnp.float32))
            else:
                tiles.append(negslot)
        blk = _block_top8(tiles)
        bv_, bi_ = [], []
        for r in range(K):
            y = jax.lax.bitcast_convert_type(blk[r], jnp.int32)
            slot = (y & 15) + g * 16
            bi_.append(slot * 8 + subl + j * BV)
            bv_.append(jax.lax.bitcast_convert_type(y & -16, jnp.float32))
        gv.append(bv_)
        gi.append(bi_)

    while len(gv) > 1:
        nv, ni = [], []
        for t in range(0, len(gv), 2):
            mv, mi = _merge_top8_kv(gv[t], gi[t], gv[t + 1], gi[t + 1])
            nv.append(mv)
            ni.append(mi)
        gv, gi = nv, ni

    pv = [pv_scr[p * 8:(p + 1) * 8, :] for p in range(K)]
    pi = [pi_scr[p * 8:(p + 1) * 8, :] for p in range(K)]
    nv, ni = _merge_top8_kv(pv, pi, gv[0], gi[0])
    for p in range(K):
        pv_scr[p * 8:(p + 1) * 8, :] = nv[p]
        pi_scr[p * 8:(p + 1) * 8, :] = ni[p]

    @pl.when(j == NBLK - 1)
    def _finalize():
        x = pv_scr[...]
        ids = pi_scr[...]
        tvs, tis = [], []
        for _ in range(K):
            m = jnp.max(x, axis=0, keepdims=True)
            sel = jnp.min(jnp.where(x == m, ids, BIGI), axis=0,
                          keepdims=True)
            tvs.append(m)
            tis.append(sel)
            x = jnp.where(ids == sel, NEG, x)
        tv = jnp.concatenate(tvs, axis=0)
        ti = jnp.concatenate(tis, axis=0)
        lse = m_scr[...] + jnp.log(s_scr[...])
        sc = prev_ref[...] + tv - lse
        beam = jax.lax.broadcasted_iota(jnp.int32, (K, LANES), 1)
        flat = beam * VOCAB + ti
        ocol = jax.lax.broadcasted_iota(jnp.int32, (1, K), 1)
        ov = jnp.zeros((1, K), jnp.float32)
        oi = jnp.zeros((1, K), jnp.int32)
        for r in range(K):
            m = jnp.max(sc, axis=(0, 1), keepdims=True)
            chosen = jnp.min(jnp.where(sc == m, flat, BIGI),
                             axis=(0, 1), keepdims=True)
            ov = jnp.where(ocol == r, m, ov)
            oi = jnp.where(ocol == r, chosen, oi)
            sc = jnp.where(flat == chosen, NEG, sc)
        vals_out[...] = ov
        ids_out[...] = oi


@functools.partial(jax.jit, static_argnames=())
def kernel(hidden, W, b, prev_log_probs):
    wt = W.T
    hidt = hidden.T
    b2 = b.reshape(NBLK, 1, BV)
    prev2 = prev_log_probs.reshape(1, BEAMS)
    vals, flat = pl.pallas_call(
        _step,
        grid=(NBLK,),
        in_specs=[
            pl.BlockSpec((HID, BEAMS), lambda j: (0, 0)),
            pl.BlockSpec((BV, HID), lambda j: (j, 0)),
            pl.BlockSpec((1, 1, BV), lambda j: (j, 0, 0)),
            pl.BlockSpec((1, BEAMS), lambda j: (0, 0)),
        ],
        out_specs=[
            pl.BlockSpec((1, K), lambda j: (0, 0)),
            pl.BlockSpec((1, K), lambda j: (0, 0)),
        ],
        out_shape=[
            jax.ShapeDtypeStruct((1, K), jnp.float32),
            jax.ShapeDtypeStruct((1, K), jnp.int32),
        ],
        scratch_shapes=[
            pltpu.VMEM((1, LANES), jnp.float32),
            pltpu.VMEM((1, LANES), jnp.float32),
            pltpu.VMEM((8 * K, LANES), jnp.float32),
            pltpu.VMEM((8 * K, LANES), jnp.int32),
        ],
        compiler_params=pltpu.CompilerParams(
            dimension_semantics=("arbitrary",),
        ),
    )(hidt, wt, b2, prev2)
    vals = vals.reshape(K)
    flat = flat.reshape(K)
    beam_ids = flat // VOCAB
    token_ids = flat % VOCAB
    return vals, beam_ids, token_ids
